# baseline (speedup 1.0000x reference)
import jax
import jax.numpy as jnp
from jax.experimental import pallas as pl

IN_CH = 32
E_DIM = 32
N_E = 1024
DEPTH = 6
BETA = 0.25

TILE = 1024


def _rvq_kernel(z_ref, cbt_ref, cb_ref, zq_ref, idx_ref, loss_ref):
    i = pl.program_id(0)

    z = z_ref[...]
    cbt = cbt_ref[...]
    cb = cb_ref[...]

    cb_sq = jnp.sum(cb * cb, axis=1)[None, :]

    iota = jax.lax.broadcasted_iota(jnp.int32, (TILE, N_E), 1)

    residual = z
    zq = jnp.zeros_like(z)
    loss_sum = jnp.zeros((), jnp.float32)
    idx_rows = []
    for _ in range(DEPTH):
        r_sq = jnp.sum(residual * residual, axis=1, keepdims=True)
        d = (r_sq + cb_sq) - 2.0 * jnp.dot(
            residual.astype(jnp.bfloat16), cbt.astype(jnp.bfloat16),
            preferred_element_type=jnp.float32,
        )
        m = jnp.min(d, axis=1, keepdims=True)
        idx = jnp.min(jnp.where(d == m, iota, N_E), axis=1)
        onehot = (iota == idx[:, None]).astype(jnp.float32)
        e = jnp.dot(
            onehot, cb,
            preferred_element_type=jnp.float32,
            precision=jax.lax.Precision.HIGHEST,
        )
        zq = zq + e
        diff = e - residual
        loss_sum = loss_sum + jnp.sum(diff * diff)
        residual = residual - e
        idx_rows.append(idx)

    zq_ref[...] = zq
    idx_ref[...] = jnp.stack(idx_rows, axis=0)

    @pl.when(i == 0)
    def _():
        loss_ref[...] = jnp.zeros((1, 1), jnp.float32)

    loss_ref[...] += loss_sum.reshape(1, 1)


def kernel(x, codebook):
    B, C, H, W = x.shape
    n = B * H * W
    z = x.transpose(0, 2, 3, 1).reshape(n, C)

    grid = (n // TILE,)
    zq, idx, loss_acc = pl.pallas_call(
        _rvq_kernel,
        grid=grid,
        in_specs=[
            pl.BlockSpec((TILE, E_DIM), lambda i: (i, 0)),
            pl.BlockSpec((E_DIM, N_E), lambda i: (0, 0)),
            pl.BlockSpec((N_E, E_DIM), lambda i: (0, 0)),
        ],
        out_specs=[
            pl.BlockSpec((TILE, E_DIM), lambda i: (i, 0)),
            pl.BlockSpec((DEPTH, TILE), lambda i: (0, i)),
            pl.BlockSpec((1, 1), lambda i: (0, 0)),
        ],
        out_shape=[
            jax.ShapeDtypeStruct((n, E_DIM), jnp.float32),
            jax.ShapeDtypeStruct((DEPTH, n), jnp.int32),
            jax.ShapeDtypeStruct((1, 1), jnp.float32),
        ],
    )(z, codebook.T, codebook)

    z_q_fold = zq.reshape(B, H, W, C).transpose(0, 3, 1, 2)
    loss = loss_acc[0, 0] * ((1.0 + BETA) / (n * E_DIM))
    return z_q_fold, loss, idx


# speedup vs baseline: 1.0243x; 1.0243x over previous
import jax
import jax.numpy as jnp
from jax.experimental import pallas as pl

IN_CH = 32
E_DIM = 32
N_E = 1024
DEPTH = 6
BETA = 0.25

TILE = 1024


def _rvq_kernel(z_ref, cbt_ref, cb_ref, zq_ref, idx_ref, loss_ref):
    i = pl.program_id(0)

    z = z_ref[...]
    cbt = cbt_ref[...]
    cb = cb_ref[...]

    cb_sq = jnp.sum(cb * cb, axis=1)[None, :]

    iota_f = jax.lax.broadcasted_iota(jnp.int32, (TILE, N_E), 1).astype(jnp.float32)

    residual = z
    zq = jnp.zeros_like(z)
    loss_sum = jnp.zeros((), jnp.float32)
    idx_rows = []
    for _ in range(DEPTH):
        r_sq = jnp.sum(residual * residual, axis=1, keepdims=True)
        d = (r_sq + cb_sq) - 2.0 * jnp.dot(
            residual.astype(jnp.bfloat16), cbt.astype(jnp.bfloat16),
            preferred_element_type=jnp.float32,
        )
        m = jnp.min(d, axis=1, keepdims=True)
        idx_f = jnp.min(
            jnp.where(d == m, iota_f, float(N_E)), axis=1, keepdims=True
        )
        onehot = (iota_f == idx_f).astype(jnp.float32)
        e = jnp.dot(
            onehot, cb,
            preferred_element_type=jnp.float32,
            precision=jax.lax.Precision.HIGHEST,
        )
        zq = zq + e
        diff = e - residual
        loss_sum = loss_sum + jnp.sum(diff * diff)
        residual = residual - e
        idx_rows.append(idx_f[:, 0])

    zq_ref[...] = zq
    idx_ref[...] = jnp.stack(idx_rows, axis=0).astype(jnp.int32)

    @pl.when(i == 0)
    def _():
        loss_ref[...] = jnp.zeros((1, 1), jnp.float32)

    loss_ref[...] += loss_sum.reshape(1, 1)


def kernel(x, codebook):
    B, C, H, W = x.shape
    n = B * H * W
    z = x.transpose(0, 2, 3, 1).reshape(n, C)

    grid = (n // TILE,)
    zq, idx, loss_acc = pl.pallas_call(
        _rvq_kernel,
        grid=grid,
        in_specs=[
            pl.BlockSpec((TILE, E_DIM), lambda i: (i, 0)),
            pl.BlockSpec((E_DIM, N_E), lambda i: (0, 0)),
            pl.BlockSpec((N_E, E_DIM), lambda i: (0, 0)),
        ],
        out_specs=[
            pl.BlockSpec((TILE, E_DIM), lambda i: (i, 0)),
            pl.BlockSpec((DEPTH, TILE), lambda i: (0, i)),
            pl.BlockSpec((1, 1), lambda i: (0, 0)),
        ],
        out_shape=[
            jax.ShapeDtypeStruct((n, E_DIM), jnp.float32),
            jax.ShapeDtypeStruct((DEPTH, n), jnp.int32),
            jax.ShapeDtypeStruct((1, 1), jnp.float32),
        ],
    )(z, codebook.T, codebook)

    z_q_fold = zq.reshape(B, H, W, C).transpose(0, 3, 1, 2)
    loss = loss_acc[0, 0] * ((1.0 + BETA) / (n * E_DIM))
    return z_q_fold, loss, idx


# speedup vs baseline: 2.0564x; 2.0076x over previous
import jax
import jax.numpy as jnp
from jax.experimental import pallas as pl

IN_CH = 32
E_DIM = 32
N_E = 1024
DEPTH = 6
BETA = 0.25

TILE = 1024


def _rvq_kernel(z_ref, cbt_ref, cb_ref, zq_ref, idx_ref, loss_ref):
    i = pl.program_id(0)

    z = z_ref[...]
    cbt = cbt_ref[...]
    cb = cb_ref[...]

    cb_sq = jnp.sum(cb * cb, axis=1)[None, :]

    cb_hi = cb.astype(jnp.bfloat16)
    r1 = cb - cb_hi.astype(jnp.float32)
    cb_lo = r1.astype(jnp.bfloat16)
    cb_lo2 = (r1 - cb_lo.astype(jnp.float32)).astype(jnp.bfloat16)

    iota_f = jax.lax.broadcasted_iota(jnp.int32, (TILE, N_E), 1).astype(jnp.float32)

    residual = z
    zq = jnp.zeros_like(z)
    loss_sum = jnp.zeros((), jnp.float32)
    idx_rows = []
    for _ in range(DEPTH):
        r_sq = jnp.sum(residual * residual, axis=1, keepdims=True)
        d = (r_sq + cb_sq) - 2.0 * jnp.dot(
            residual.astype(jnp.bfloat16), cbt.astype(jnp.bfloat16),
            preferred_element_type=jnp.float32,
        )
        m = jnp.min(d, axis=1, keepdims=True)
        idx_f = jnp.min(
            jnp.where(d == m, iota_f, float(N_E)), axis=1, keepdims=True
        )
        onehot = (iota_f == idx_f).astype(jnp.bfloat16)
        e = jnp.dot(onehot, cb_hi, preferred_element_type=jnp.float32)
        e = e + jnp.dot(onehot, cb_lo, preferred_element_type=jnp.float32)
        e = e + jnp.dot(onehot, cb_lo2, preferred_element_type=jnp.float32)
        zq = zq + e
        diff = e - residual
        loss_sum = loss_sum + jnp.sum(diff * diff)
        residual = residual - e
        idx_rows.append(idx_f[:, 0])

    zq_ref[...] = zq
    idx_ref[...] = jnp.stack(idx_rows, axis=0).astype(jnp.int32)

    @pl.when(i == 0)
    def _():
        loss_ref[...] = jnp.zeros((1, 1), jnp.float32)

    loss_ref[...] += loss_sum.reshape(1, 1)


def kernel(x, codebook):
    B, C, H, W = x.shape
    n = B * H * W
    z = x.transpose(0, 2, 3, 1).reshape(n, C)

    grid = (n // TILE,)
    zq, idx, loss_acc = pl.pallas_call(
        _rvq_kernel,
        grid=grid,
        in_specs=[
            pl.BlockSpec((TILE, E_DIM), lambda i: (i, 0)),
            pl.BlockSpec((E_DIM, N_E), lambda i: (0, 0)),
            pl.BlockSpec((N_E, E_DIM), lambda i: (0, 0)),
        ],
        out_specs=[
            pl.BlockSpec((TILE, E_DIM), lambda i: (i, 0)),
            pl.BlockSpec((DEPTH, TILE), lambda i: (0, i)),
            pl.BlockSpec((1, 1), lambda i: (0, 0)),
        ],
        out_shape=[
            jax.ShapeDtypeStruct((n, E_DIM), jnp.float32),
            jax.ShapeDtypeStruct((DEPTH, n), jnp.int32),
            jax.ShapeDtypeStruct((1, 1), jnp.float32),
        ],
    )(z, codebook.T, codebook)

    z_q_fold = zq.reshape(B, H, W, C).transpose(0, 3, 1, 2)
    loss = loss_acc[0, 0] * ((1.0 + BETA) / (n * E_DIM))
    return z_q_fold, loss, idx


# speedup vs baseline: 3.1146x; 1.5146x over previous
import jax
import jax.numpy as jnp
from jax.experimental import pallas as pl

IN_CH = 32
E_DIM = 32
N_E = 1024
DEPTH = 6
BETA = 0.25

TILE = 1024
HALF = TILE // 2


def _rvq_kernel(z_ref, cbt_ref, cb_ref, zq_ref, idx_ref, loss_ref):
    i = pl.program_id(0)

    cbt = cbt_ref[...]
    cb = cb_ref[...]

    cb_sq = jnp.sum(cb * cb, axis=1)[None, :]
    cbt_m2 = (cbt * -2.0).astype(jnp.bfloat16)

    cb_hi = cb.astype(jnp.bfloat16)
    r1 = cb - cb_hi.astype(jnp.float32)
    cb_lo = r1.astype(jnp.bfloat16)
    cb_lo2 = (r1 - cb_lo.astype(jnp.float32)).astype(jnp.bfloat16)

    iota_f = jax.lax.broadcasted_iota(jnp.int32, (HALF, N_E), 1).astype(jnp.float32)

    def level(residual):
        r_sq = jnp.sum(residual * residual, axis=1, keepdims=True)
        d = (r_sq + cb_sq) + jnp.dot(
            residual.astype(jnp.bfloat16), cbt_m2,
            preferred_element_type=jnp.float32,
        )
        m = jnp.min(d, axis=1, keepdims=True)
        idx_f = jnp.min(
            jnp.where(d == m, iota_f, float(N_E)), axis=1, keepdims=True
        )
        onehot = (iota_f == idx_f).astype(jnp.bfloat16)
        e = jnp.dot(onehot, cb_hi, preferred_element_type=jnp.float32)
        e = e + jnp.dot(onehot, cb_lo, preferred_element_type=jnp.float32)
        e = e + jnp.dot(onehot, cb_lo2, preferred_element_type=jnp.float32)
        return e, idx_f

    res = [z_ref[0:HALF, :], z_ref[HALF:TILE, :]]
    zq = [jnp.zeros((HALF, E_DIM), jnp.float32) for _ in range(2)]
    loss_sum = jnp.zeros((), jnp.float32)
    idx_rows = [[], []]
    for _ in range(DEPTH):
        for h in range(2):
            e, idx_f = level(res[h])
            zq[h] = zq[h] + e
            diff = e - res[h]
            loss_sum = loss_sum + jnp.sum(diff * diff)
            res[h] = res[h] - e
            idx_rows[h].append(idx_f[:, 0])

    zq_ref[0:HALF, :] = zq[0]
    zq_ref[HALF:TILE, :] = zq[1]
    idx_ref[:, 0:HALF] = jnp.stack(idx_rows[0], axis=0).astype(jnp.int32)
    idx_ref[:, HALF:TILE] = jnp.stack(idx_rows[1], axis=0).astype(jnp.int32)

    @pl.when(i == 0)
    def _():
        loss_ref[...] = jnp.zeros((1, 1), jnp.float32)

    loss_ref[...] += loss_sum.reshape(1, 1)


def kernel(x, codebook):
    B, C, H, W = x.shape
    n = B * H * W
    z = x.transpose(0, 2, 3, 1).reshape(n, C)

    grid = (n // TILE,)
    zq, idx, loss_acc = pl.pallas_call(
        _rvq_kernel,
        grid=grid,
        in_specs=[
            pl.BlockSpec((TILE, E_DIM), lambda i: (i, 0)),
            pl.BlockSpec((E_DIM, N_E), lambda i: (0, 0)),
            pl.BlockSpec((N_E, E_DIM), lambda i: (0, 0)),
        ],
        out_specs=[
            pl.BlockSpec((TILE, E_DIM), lambda i: (i, 0)),
            pl.BlockSpec((DEPTH, TILE), lambda i: (0, i)),
            pl.BlockSpec((1, 1), lambda i: (0, 0)),
        ],
        out_shape=[
            jax.ShapeDtypeStruct((n, E_DIM), jnp.float32),
            jax.ShapeDtypeStruct((DEPTH, n), jnp.int32),
            jax.ShapeDtypeStruct((1, 1), jnp.float32),
        ],
    )(z, codebook.T, codebook)

    z_q_fold = zq.reshape(B, H, W, C).transpose(0, 3, 1, 2)
    loss = loss_acc[0, 0] * ((1.0 + BETA) / (n * E_DIM))
    return z_q_fold, loss, idx


# speedup vs baseline: 3.1211x; 1.0021x over previous
import jax
import jax.numpy as jnp
from jax.experimental import pallas as pl
from jax.experimental.pallas import tpu as pltpu

IN_CH = 32
E_DIM = 32
N_E = 1024
DEPTH = 6
BETA = 0.25

TILE = 1024
HALF = TILE // 2


def _rvq_kernel(z_ref, cbt_ref, cb_ref, zq_ref, idx_ref, loss_ref):
    cbt = cbt_ref[...]
    cb = cb_ref[...]

    cb_sq = jnp.sum(cb * cb, axis=1)[None, :]
    cbt_m2 = (cbt * -2.0).astype(jnp.bfloat16)

    cb_hi = cb.astype(jnp.bfloat16)
    r1 = cb - cb_hi.astype(jnp.float32)
    cb_lo = r1.astype(jnp.bfloat16)
    cb_lo2 = (r1 - cb_lo.astype(jnp.float32)).astype(jnp.bfloat16)

    iota_f = jax.lax.broadcasted_iota(jnp.int32, (HALF, N_E), 1).astype(jnp.float32)

    def level(residual):
        r_sq = jnp.sum(residual * residual, axis=1, keepdims=True)
        d = (r_sq + cb_sq) + jnp.dot(
            residual.astype(jnp.bfloat16), cbt_m2,
            preferred_element_type=jnp.float32,
        )
        m = jnp.min(d, axis=1, keepdims=True)
        idx_f = jnp.min(
            jnp.where(d == m, iota_f, float(N_E)), axis=1, keepdims=True
        )
        onehot = (iota_f == idx_f).astype(jnp.bfloat16)
        e = jnp.dot(onehot, cb_hi, preferred_element_type=jnp.float32)
        e = e + jnp.dot(onehot, cb_lo, preferred_element_type=jnp.float32)
        e = e + jnp.dot(onehot, cb_lo2, preferred_element_type=jnp.float32)
        return e, idx_f

    res = [z_ref[0:HALF, :], z_ref[HALF:TILE, :]]
    zq = [jnp.zeros((HALF, E_DIM), jnp.float32) for _ in range(2)]
    loss_sum = jnp.zeros((), jnp.float32)
    idx_rows = [[], []]
    for _ in range(DEPTH):
        for h in range(2):
            e, idx_f = level(res[h])
            zq[h] = zq[h] + e
            diff = e - res[h]
            loss_sum = loss_sum + jnp.sum(diff * diff)
            res[h] = res[h] - e
            idx_rows[h].append(idx_f[:, 0])

    zq_ref[0:HALF, :] = zq[0]
    zq_ref[HALF:TILE, :] = zq[1]
    idx_ref[:, 0:HALF] = jnp.stack(idx_rows[0], axis=0).astype(jnp.int32)
    idx_ref[:, HALF:TILE] = jnp.stack(idx_rows[1], axis=0).astype(jnp.int32)

    loss_ref[...] = jnp.full((8, 128), loss_sum, jnp.float32)


def kernel(x, codebook):
    B, C, H, W = x.shape
    n = B * H * W
    z = x.transpose(0, 2, 3, 1).reshape(n, C)

    grid = (n // TILE,)
    zq, idx, loss_acc = pl.pallas_call(
        _rvq_kernel,
        grid=grid,
        in_specs=[
            pl.BlockSpec((TILE, E_DIM), lambda i: (i, 0)),
            pl.BlockSpec((E_DIM, N_E), lambda i: (0, 0)),
            pl.BlockSpec((N_E, E_DIM), lambda i: (0, 0)),
        ],
        out_specs=[
            pl.BlockSpec((TILE, E_DIM), lambda i: (i, 0)),
            pl.BlockSpec((DEPTH, TILE), lambda i: (0, i)),
            pl.BlockSpec((8, 128), lambda i: (i, 0)),
        ],
        out_shape=[
            jax.ShapeDtypeStruct((n, E_DIM), jnp.float32),
            jax.ShapeDtypeStruct((DEPTH, n), jnp.int32),
            jax.ShapeDtypeStruct((n // TILE * 8, 128), jnp.float32),
        ],
        compiler_params=pltpu.CompilerParams(
            dimension_semantics=("parallel",),
        ),
    )(z, codebook.T, codebook)

    z_q_fold = zq.reshape(B, H, W, C).transpose(0, 3, 1, 2)
    loss = jnp.sum(loss_acc[::8, 0]) * ((1.0 + BETA) / (n * E_DIM))
    return z_q_fold, loss, idx


# speedup vs baseline: 3.2487x; 1.0409x over previous
import jax
import jax.numpy as jnp
from jax.experimental import pallas as pl
from jax.experimental.pallas import tpu as pltpu

IN_CH = 32
E_DIM = 32
N_E = 1024
DEPTH = 6
BETA = 0.25

TILE = 1024
HALF = TILE // 2


def _rvq_kernel(z_ref, cbt_ref, cb_ref, zq_ref, idx_ref, loss_ref):
    cbt = cbt_ref[...]
    cb = cb_ref[...]

    cb_sq = jnp.sum(cb * cb, axis=1)[None, :]
    cbt_m2 = (cbt * -2.0).astype(jnp.bfloat16)

    cb_hi = cb.astype(jnp.bfloat16)
    r1 = cb - cb_hi.astype(jnp.float32)
    cb_lo = r1.astype(jnp.bfloat16)
    cb_lo2 = (r1 - cb_lo.astype(jnp.float32)).astype(jnp.bfloat16)
    cb3 = jnp.concatenate([cb_hi, cb_lo, cb_lo2], axis=1)

    iota_f = jax.lax.broadcasted_iota(jnp.int32, (HALF, N_E), 1).astype(jnp.float32)

    def level(residual):
        r_sq = jnp.sum(residual * residual, axis=1, keepdims=True)
        d = (r_sq + cb_sq) + jnp.dot(
            residual.astype(jnp.bfloat16), cbt_m2,
            preferred_element_type=jnp.float32,
        )
        m = jnp.min(d, axis=1, keepdims=True)
        idx_f = jnp.min(
            jnp.where(d == m, iota_f, float(N_E)), axis=1, keepdims=True
        )
        onehot = (iota_f == idx_f).astype(jnp.bfloat16)
        g = jnp.dot(onehot, cb3, preferred_element_type=jnp.float32)
        e = (g[:, 0:E_DIM] + g[:, E_DIM:2 * E_DIM]) + g[:, 2 * E_DIM:3 * E_DIM]
        return e, idx_f

    res = [z_ref[0:HALF, :], z_ref[HALF:TILE, :]]
    zq = [jnp.zeros((HALF, E_DIM), jnp.float32) for _ in range(2)]
    loss_sum = jnp.zeros((), jnp.float32)
    idx_rows = [[], []]
    for _ in range(DEPTH):
        for h in range(2):
            e, idx_f = level(res[h])
            zq[h] = zq[h] + e
            diff = e - res[h]
            loss_sum = loss_sum + jnp.sum(diff * diff)
            res[h] = res[h] - e
            idx_rows[h].append(idx_f[:, 0])

    zq_ref[0:HALF, :] = zq[0]
    zq_ref[HALF:TILE, :] = zq[1]
    idx_ref[:, 0:HALF] = jnp.stack(idx_rows[0], axis=0).astype(jnp.int32)
    idx_ref[:, HALF:TILE] = jnp.stack(idx_rows[1], axis=0).astype(jnp.int32)

    loss_ref[...] = jnp.full((8, 128), loss_sum, jnp.float32)


def kernel(x, codebook):
    B, C, H, W = x.shape
    n = B * H * W
    z = x.transpose(0, 2, 3, 1).reshape(n, C)

    grid = (n // TILE,)
    zq, idx, loss_acc = pl.pallas_call(
        _rvq_kernel,
        grid=grid,
        in_specs=[
            pl.BlockSpec((TILE, E_DIM), lambda i: (i, 0)),
            pl.BlockSpec((E_DIM, N_E), lambda i: (0, 0)),
            pl.BlockSpec((N_E, E_DIM), lambda i: (0, 0)),
        ],
        out_specs=[
            pl.BlockSpec((TILE, E_DIM), lambda i: (i, 0)),
            pl.BlockSpec((DEPTH, TILE), lambda i: (0, i)),
            pl.BlockSpec((8, 128), lambda i: (i, 0)),
        ],
        out_shape=[
            jax.ShapeDtypeStruct((n, E_DIM), jnp.float32),
            jax.ShapeDtypeStruct((DEPTH, n), jnp.int32),
            jax.ShapeDtypeStruct((n // TILE * 8, 128), jnp.float32),
        ],
        compiler_params=pltpu.CompilerParams(
            dimension_semantics=("parallel",),
        ),
    )(z, codebook.T, codebook)

    z_q_fold = zq.reshape(B, H, W, C).transpose(0, 3, 1, 2)
    loss = jnp.sum(loss_acc[::8, 0]) * ((1.0 + BETA) / (n * E_DIM))
    return z_q_fold, loss, idx


# speedup vs baseline: 3.2519x; 1.0010x over previous
import jax
import jax.numpy as jnp
from jax.experimental import pallas as pl
from jax.experimental.pallas import tpu as pltpu

IN_CH = 32
E_DIM = 32
N_E = 1024
DEPTH = 6
BETA = 0.25

TILE = 1024
HALF = TILE // 2


def _rvq_kernel(z_ref, cbt_ref, cb_ref, zq_ref, idx_ref, loss_ref):
    cbt = cbt_ref[...]
    cb = cb_ref[...]

    cb_sq = jnp.sum(cb * cb, axis=1)[None, :]
    cbt_m2 = (cbt * -2.0).astype(jnp.bfloat16)

    cb_hi = cb.astype(jnp.bfloat16)
    r1 = cb - cb_hi.astype(jnp.float32)
    cb_lo = r1.astype(jnp.bfloat16)
    cb_lo2 = (r1 - cb_lo.astype(jnp.float32)).astype(jnp.bfloat16)
    cb3 = jnp.concatenate([cb_hi, cb_lo, cb_lo2], axis=1)

    iota_f = jax.lax.broadcasted_iota(jnp.int32, (1, N_E), 1).astype(jnp.float32)

    def level(residual):
        r_sq = jnp.sum(residual * residual, axis=1, keepdims=True)
        d = (r_sq + cb_sq) + jnp.dot(
            residual.astype(jnp.bfloat16), cbt_m2,
            preferred_element_type=jnp.float32,
        )
        m = jnp.min(d, axis=1, keepdims=True)
        idx_f = jnp.min(
            jnp.where(d == m, iota_f, float(N_E)), axis=1, keepdims=True
        )
        onehot = (iota_f == idx_f).astype(jnp.bfloat16)
        g = jnp.dot(onehot, cb3, preferred_element_type=jnp.float32)
        e = (g[:, 0:E_DIM] + g[:, E_DIM:2 * E_DIM]) + g[:, 2 * E_DIM:3 * E_DIM]
        return e, idx_f

    res = [z_ref[0:HALF, :], z_ref[HALF:TILE, :]]
    zq = [jnp.zeros((HALF, E_DIM), jnp.float32) for _ in range(2)]
    loss_sum = jnp.zeros((), jnp.float32)
    idx_rows = [[], []]
    for _ in range(DEPTH):
        for h in range(2):
            e, idx_f = level(res[h])
            zq[h] = zq[h] + e
            diff = e - res[h]
            loss_sum = loss_sum + jnp.sum(diff * diff)
            res[h] = res[h] - e
            idx_rows[h].append(idx_f[:, 0])

    zq_ref[0:HALF, :] = zq[0]
    zq_ref[HALF:TILE, :] = zq[1]
    idx_ref[:, 0:HALF] = jnp.stack(idx_rows[0], axis=0).astype(jnp.int32)
    idx_ref[:, HALF:TILE] = jnp.stack(idx_rows[1], axis=0).astype(jnp.int32)

    loss_ref[...] = jnp.full((8, 128), loss_sum, jnp.float32)


def kernel(x, codebook):
    B, C, H, W = x.shape
    n = B * H * W
    z = x.transpose(0, 2, 3, 1).reshape(n, C)

    grid = (n // TILE,)
    zq, idx, loss_acc = pl.pallas_call(
        _rvq_kernel,
        grid=grid,
        in_specs=[
            pl.BlockSpec((TILE, E_DIM), lambda i: (i, 0)),
            pl.BlockSpec((E_DIM, N_E), lambda i: (0, 0)),
            pl.BlockSpec((N_E, E_DIM), lambda i: (0, 0)),
        ],
        out_specs=[
            pl.BlockSpec((TILE, E_DIM), lambda i: (i, 0)),
            pl.BlockSpec((DEPTH, TILE), lambda i: (0, i)),
            pl.BlockSpec((8, 128), lambda i: (i, 0)),
        ],
        out_shape=[
            jax.ShapeDtypeStruct((n, E_DIM), jnp.float32),
            jax.ShapeDtypeStruct((DEPTH, n), jnp.int32),
            jax.ShapeDtypeStruct((n // TILE * 8, 128), jnp.float32),
        ],
        compiler_params=pltpu.CompilerParams(
            dimension_semantics=("parallel",),
        ),
    )(z, codebook.T, codebook)

    z_q_fold = zq.reshape(B, H, W, C).transpose(0, 3, 1, 2)
    loss = jnp.sum(loss_acc[::8, 0]) * ((1.0 + BETA) / (n * E_DIM))
    return z_q_fold, loss, idx


# speedup vs baseline: 3.7526x; 1.1540x over previous
import jax
import jax.numpy as jnp
from jax.experimental import pallas as pl
from jax.experimental.pallas import tpu as pltpu

IN_CH = 32
E_DIM = 32
N_E = 1024
DEPTH = 6
BETA = 0.25

TILE = 1024
NSTREAM = 4
SROWS = TILE // NSTREAM


def _rvq_kernel(z_ref, cbt_ref, cb_ref, zq_ref, idx_ref, loss_ref):
    cbt = cbt_ref[...]
    cb = cb_ref[...]

    cb_sq = jnp.sum(cb * cb, axis=1)[None, :]
    cbt_m2 = (cbt * -2.0).astype(jnp.bfloat16)

    cb_hi = cb.astype(jnp.bfloat16)
    r1 = cb - cb_hi.astype(jnp.float32)
    cb_lo = r1.astype(jnp.bfloat16)
    cb_lo2 = (r1 - cb_lo.astype(jnp.float32)).astype(jnp.bfloat16)
    cb3 = jnp.concatenate([cb_hi, cb_lo, cb_lo2], axis=1)

    iota_f = jax.lax.broadcasted_iota(jnp.int32, (1, N_E), 1).astype(jnp.float32)

    def level(residual):
        r_sq = jnp.sum(residual * residual, axis=1, keepdims=True)
        d = (r_sq + cb_sq) + jnp.dot(
            residual.astype(jnp.bfloat16), cbt_m2,
            preferred_element_type=jnp.float32,
        )
        m = jnp.min(d, axis=1, keepdims=True)
        idx_f = jnp.min(
            jnp.where(d == m, iota_f, float(N_E)), axis=1, keepdims=True
        )
        onehot = (iota_f == idx_f).astype(jnp.bfloat16)
        g = jnp.dot(onehot, cb3, preferred_element_type=jnp.float32)
        e = (g[:, 0:E_DIM] + g[:, E_DIM:2 * E_DIM]) + g[:, 2 * E_DIM:3 * E_DIM]
        return e, idx_f

    res = [z_ref[h * SROWS:(h + 1) * SROWS, :] for h in range(NSTREAM)]
    zq = [jnp.zeros((SROWS, E_DIM), jnp.float32) for _ in range(NSTREAM)]
    loss_sum = jnp.zeros((), jnp.float32)
    idx_rows = [[] for _ in range(NSTREAM)]
    for _ in range(DEPTH):
        for h in range(NSTREAM):
            e, idx_f = level(res[h])
            zq[h] = zq[h] + e
            diff = e - res[h]
            loss_sum = loss_sum + jnp.sum(diff * diff)
            res[h] = res[h] - e
            idx_rows[h].append(idx_f[:, 0])

    for h in range(NSTREAM):
        zq_ref[h * SROWS:(h + 1) * SROWS, :] = zq[h]
        idx_ref[:, h * SROWS:(h + 1) * SROWS] = jnp.stack(
            idx_rows[h], axis=0).astype(jnp.int32)

    loss_ref[...] = jnp.full((8, 128), loss_sum, jnp.float32)


def kernel(x, codebook):
    B, C, H, W = x.shape
    n = B * H * W
    z = x.transpose(0, 2, 3, 1).reshape(n, C)

    grid = (n // TILE,)
    zq, idx, loss_acc = pl.pallas_call(
        _rvq_kernel,
        grid=grid,
        in_specs=[
            pl.BlockSpec((TILE, E_DIM), lambda i: (i, 0)),
            pl.BlockSpec((E_DIM, N_E), lambda i: (0, 0)),
            pl.BlockSpec((N_E, E_DIM), lambda i: (0, 0)),
        ],
        out_specs=[
            pl.BlockSpec((TILE, E_DIM), lambda i: (i, 0)),
            pl.BlockSpec((DEPTH, TILE), lambda i: (0, i)),
            pl.BlockSpec((8, 128), lambda i: (i, 0)),
        ],
        out_shape=[
            jax.ShapeDtypeStruct((n, E_DIM), jnp.float32),
            jax.ShapeDtypeStruct((DEPTH, n), jnp.int32),
            jax.ShapeDtypeStruct((n // TILE * 8, 128), jnp.float32),
        ],
        compiler_params=pltpu.CompilerParams(
            dimension_semantics=("parallel",),
        ),
    )(z, codebook.T, codebook)

    z_q_fold = zq.reshape(B, H, W, C).transpose(0, 3, 1, 2)
    loss = jnp.sum(loss_acc[::8, 0]) * ((1.0 + BETA) / (n * E_DIM))
    return z_q_fold, loss, idx
